# baseline (speedup 1.0000x reference)
import jax
import jax.numpy as jnp
from jax.experimental import pallas as pl
from jax.experimental.pallas import tpu as pltpu

N_NODES = 10000
D = 128


def _hfn_kernel(x_ref, w_ref, b_ref, h_ref, n_ref):
    h = jnp.tanh(
        jax.lax.dot_general(x_ref[...], w_ref[...], (((1,), (1,)), ((), ())),
                            preferred_element_type=jnp.float32)
        + b_ref[...][None, :]
    )
    h_ref[...] = h
    n_ref[...] = jnp.sqrt(jnp.sum(h * h, axis=1, keepdims=True))


def _hfn(x, W, b):
    return pl.pallas_call(
        _hfn_kernel,
        out_shape=(
            jax.ShapeDtypeStruct((N_NODES, D), jnp.float32),
            jax.ShapeDtypeStruct((N_NODES, 1), jnp.float32),
        ),
    )(x, W, b)


def kernel(x, mask, edge_index, W, b, alpha):
    N = x.shape[0]
    h, n = _hfn(x, W, b)
    n = n[:, 0]
    src = edge_index[0]
    dst = edge_index[1]
    a = jnp.take(h, src, axis=0)
    bb = jnp.take(h, dst, axis=0)
    dot = jnp.sum(a * bb, axis=-1)
    cos = dot / jnp.maximum(n[src] * n[dst], 1e-8)
    edge_weights = jax.nn.relu(cos)
    fill = jax.nn.relu(mask)
    is_loop = (src == dst).astype(jnp.float32)
    has_loop = jax.ops.segment_sum(is_loop, src, num_segments=N) > 0
    loop_w = jnp.where(has_loop, 0.0, 1.0)
    ar = jnp.arange(N, dtype=src.dtype)
    row = jnp.concatenate([src, ar])
    col = jnp.concatenate([dst, ar])
    w = jnp.concatenate([edge_weights, loop_w])
    deg = jax.ops.segment_sum(w, col, num_segments=N)
    safe_deg = jnp.where(deg > 0, deg, 1.0)
    dis = jnp.where(deg > 0, 1.0 / jnp.sqrt(safe_deg), 0.0)
    norm = dis[row] * w * dis[col]
    out = fill
    for _ in range(5):
        msg = jnp.take(out, row, axis=0) * norm[:, None]
        out = jax.ops.segment_sum(msg, col, num_segments=N)
        out = out * (1.0 - alpha) + alpha * fill
    return (out, edge_weights)
